# Initial kernel scaffold; baseline (speedup 1.0000x reference)
import jax
import jax.numpy as jnp
from jax.experimental import pallas as pl


def kernel(x, edge_index, W1, b1, W2, b2):
    raise NotImplementedError("write your pallas kernel here")


# speedup vs baseline: 143.4358x; 143.4358x over previous
import functools

import jax
import jax.numpy as jnp
from jax import lax
from jax.experimental import pallas as pl
from jax.experimental.pallas import tpu as pltpu
from jax.experimental.pallas import tpu_sc as plsc

_N = 100000
_E = 6400000
_RB = 128
_ROWS = 16
_CH = _RB * _ROWS
_NCH = _E // _CH
_EROWS = _E // _RB
_NW = 32
_NITER = -(-_NCH // _NW)
_NPAD = 100096
_RPT = _NPAD // 128
_SL = _NPAD // 16

_mesh = plsc.VectorSubcoreMesh(core_axis_name="c", subcore_axis_name="s")


@functools.partial(
    pl.kernel,
    out_type=jax.ShapeDtypeStruct((2 * _NPAD,), jnp.float32),
    mesh=_mesh,
    scratch_types=[
        pltpu.VMEM((_ROWS, _RB), jnp.int32),
        pltpu.VMEM((_RB,), jnp.float32),
        pltpu.VMEM((_SL,), jnp.float32),
        pltpu.VMEM_SHARED((_NPAD,), jnp.float32),
    ],
)
def _deg_pass(edges, out, dstbuf, ones, zbuf, acc):
    c = lax.axis_index("c")
    s = lax.axis_index("s")
    w = s * 2 + c

    def _fill_zero(i, carry):
        zbuf[pl.ds(i * 16, 16)] = jnp.zeros((16,), jnp.float32)
        return carry

    lax.fori_loop(0, _SL // 16, _fill_zero, 0)

    def _fill_one(i, carry):
        ones[pl.ds(i * 16, 16)] = jnp.full((16,), 1.0, jnp.float32)
        return carry

    lax.fori_loop(0, _RB // 16, _fill_one, 0)
    pltpu.sync_copy(zbuf, acc.at[pl.ds(s * _SL, _SL)])
    plsc.subcore_barrier()

    def _chunk(i, carry):
        cid = w + i * _NW

        @pl.when(cid < _NCH)
        def _():
            pltpu.sync_copy(edges.at[1, pl.ds(cid * _ROWS, _ROWS), :], dstbuf)
            for r in range(_ROWS):
                pltpu.sync_copy(ones, acc.at[dstbuf.at[r]], add=True)

        return carry

    lax.fori_loop(0, _NITER, _chunk, 0)
    plsc.subcore_barrier()
    pltpu.sync_copy(acc.at[pl.ds(s * _SL, _SL)], zbuf)
    pltpu.sync_copy(zbuf, out.at[pl.ds(c * _NPAD + s * _SL, _SL)])


@functools.partial(
    pl.kernel,
    out_type=jax.ShapeDtypeStruct((2 * _NPAD,), jnp.float32),
    mesh=_mesh,
    scratch_types=[
        pltpu.VMEM((_ROWS, _RB), jnp.int32),
        pltpu.VMEM((_ROWS, _RB), jnp.int32),
        pltpu.VMEM((_RB,), jnp.float32),
        pltpu.VMEM((_SL,), jnp.float32),
        pltpu.VMEM_SHARED((_NPAD,), jnp.float32),
        pltpu.VMEM_SHARED((_NPAD,), jnp.float32),
    ],
)
def _agg_pass(edges, u, out, srcbuf, dstbuf, vals, zbuf, utab, acc):
    c = lax.axis_index("c")
    s = lax.axis_index("s")
    w = s * 2 + c

    def _fill_zero(i, carry):
        zbuf[pl.ds(i * 16, 16)] = jnp.zeros((16,), jnp.float32)
        return carry

    lax.fori_loop(0, _SL // 16, _fill_zero, 0)
    pltpu.sync_copy(zbuf, acc.at[pl.ds(s * _SL, _SL)])
    pltpu.sync_copy(u.at[pl.ds(s * _SL, _SL)], zbuf)
    pltpu.sync_copy(zbuf, utab.at[pl.ds(s * _SL, _SL)])
    plsc.subcore_barrier()

    def _chunk(i, carry):
        cid = w + i * _NW

        @pl.when(cid < _NCH)
        def _():
            pltpu.sync_copy(edges.at[0, pl.ds(cid * _ROWS, _ROWS), :], srcbuf)
            pltpu.sync_copy(edges.at[1, pl.ds(cid * _ROWS, _ROWS), :], dstbuf)
            for r in range(_ROWS):
                pltpu.sync_copy(utab.at[srcbuf.at[r]], vals)
                pltpu.sync_copy(vals, acc.at[dstbuf.at[r]], add=True)

        return carry

    lax.fori_loop(0, _NITER, _chunk, 0)
    plsc.subcore_barrier()
    pltpu.sync_copy(acc.at[pl.ds(s * _SL, _SL)], zbuf)
    pltpu.sync_copy(zbuf, out.at[pl.ds(c * _NPAD + s * _SL, _SL)])


def _e1_body(parts, xp, u1, dinv, xod):
    deg = parts[0] + parts[1] + 1.0
    di = lax.rsqrt(deg)
    dinv[...] = di
    u1[...] = di * xp[...]
    xod[...] = xp[...] * di * di


_e1 = pl.pallas_call(
    _e1_body,
    out_shape=(
        jax.ShapeDtypeStruct((_RPT, 128), jnp.float32),
        jax.ShapeDtypeStruct((_RPT, 128), jnp.float32),
        jax.ShapeDtypeStruct((_RPT, 128), jnp.float32),
    ),
)


def _e2_body(coef, parts, dinv, xod, u2, selfout):
    di = dinv[...]
    agg1 = di * (parts[0] + parts[1]) + xod[...]
    h1a = jnp.maximum(agg1 * coef[0] + coef[2], 0.0)
    h1b = jnp.maximum(agg1 * coef[1] + coef[3], 0.0)
    h2 = h1a * coef[4] + h1b * coef[5]
    u2[...] = di * h2
    selfout[...] = h2 * di * di + coef[6]


_e2 = pl.pallas_call(
    _e2_body,
    in_specs=[
        pl.BlockSpec(memory_space=pltpu.SMEM),
        pl.BlockSpec(memory_space=pltpu.VMEM),
        pl.BlockSpec(memory_space=pltpu.VMEM),
        pl.BlockSpec(memory_space=pltpu.VMEM),
    ],
    out_shape=(
        jax.ShapeDtypeStruct((_RPT, 128), jnp.float32),
        jax.ShapeDtypeStruct((_RPT, 128), jnp.float32),
    ),
)


def _e3_body(parts, dinv, selfout, out):
    out[...] = dinv[...] * (parts[0] + parts[1]) + selfout[...]


_e3 = pl.pallas_call(
    _e3_body,
    out_shape=jax.ShapeDtypeStruct((_RPT, 128), jnp.float32),
)


def kernel(x, edge_index, W1, b1, W2, b2):
    xf = x.reshape(_N).astype(jnp.float32)
    xp = jnp.pad(xf, (0, _NPAD - _N)).reshape(_RPT, 128)
    ei = edge_index.astype(jnp.int32).reshape(2, _EROWS, _RB)
    coef = jnp.stack(
        [W1[0, 0], W1[0, 1], b1[0], b1[1], W2[0, 0], W2[1, 0], b2[0],
         jnp.float32(0.0)]
    )
    parts0 = _deg_pass(ei)
    u1, dinv, xod = _e1(parts0.reshape(2, _RPT, 128), xp)
    parts1 = _agg_pass(ei, u1.reshape(_NPAD))
    u2, selfout = _e2(coef, parts1.reshape(2, _RPT, 128), dinv, xod)
    parts2 = _agg_pass(ei, u2.reshape(_NPAD))
    outp = _e3(parts2.reshape(2, _RPT, 128), dinv, selfout)
    return outp.reshape(_NPAD)[:_N].reshape(_N, 1)


# speedup vs baseline: 304.8733x; 2.1255x over previous
import functools

import jax
import jax.numpy as jnp
from jax import lax
from jax.experimental import pallas as pl
from jax.experimental.pallas import tpu as pltpu
from jax.experimental.pallas import tpu_sc as plsc

_N = 100000
_E = 6400000
_RB = 128
_ROWS = 16
_CH = _RB * _ROWS
_NCH = _E // _CH
_EROWS = _E // _RB
_NW = 32
_NITER = -(-_NCH // _NW)
_NPAD = 100096
_RPT = _NPAD // 128
_SL = _NPAD // 16

_mesh = plsc.VectorSubcoreMesh(core_axis_name="c", subcore_axis_name="s")


@functools.partial(
    pl.kernel,
    out_type=jax.ShapeDtypeStruct((2 * _NPAD,), jnp.float32),
    mesh=_mesh,
    scratch_types=[
        pltpu.VMEM((2, _ROWS, _RB), jnp.int32),
        pltpu.VMEM((_RB,), jnp.float32),
        pltpu.VMEM((_SL,), jnp.float32),
        pltpu.VMEM_SHARED((_NPAD,), jnp.float32),
        pltpu.SemaphoreType.DMA,
        pltpu.SemaphoreType.DMA,
        pltpu.SemaphoreType.DMA,
    ],
)
def _deg_pass(edges, out, dstbuf, ones, zbuf, acc, esem0, esem1, ssem):
    c = lax.axis_index("c")
    s = lax.axis_index("s")
    w = s * 2 + c
    esems = (esem0, esem1)

    def _fill_zero(i, carry):
        zbuf[pl.ds(i * 16, 16)] = jnp.zeros((16,), jnp.float32)
        return carry

    lax.fori_loop(0, _SL // 16, _fill_zero, 0)

    def _fill_one(i, carry):
        ones[pl.ds(i * 16, 16)] = jnp.full((16,), 1.0, jnp.float32)
        return carry

    lax.fori_loop(0, _RB // 16, _fill_one, 0)
    pltpu.sync_copy(zbuf, acc.at[pl.ds(s * _SL, _SL)])
    plsc.subcore_barrier()

    def _edges_dma(cid, b):
        return pltpu.make_async_copy(
            edges.at[1, pl.ds(cid * _ROWS, _ROWS), :], dstbuf.at[b], esems[b])

    for b in range(2):
        cid = w + b * _NW

        @pl.when(cid < _NCH)
        def _():
            _edges_dma(cid, b).start()

    def _chunk(j, carry):
        for b in range(2):
            cid = w + (2 * j + b) * _NW

            @pl.when(cid < _NCH)
            def _():
                _edges_dma(cid, b).wait()
                descs = [
                    pltpu.async_copy(ones, acc.at[dstbuf.at[b, r]], ssem,
                                     add=True)
                    for r in range(_ROWS)
                ]
                for d in descs:
                    d.wait()
                nid = cid + 2 * _NW

                @pl.when(nid < _NCH)
                def _():
                    _edges_dma(nid, b).start()

        return carry

    lax.fori_loop(0, (_NITER + 1) // 2, _chunk, 0)
    plsc.subcore_barrier()
    pltpu.sync_copy(acc.at[pl.ds(s * _SL, _SL)], zbuf)
    pltpu.sync_copy(zbuf, out.at[pl.ds(c * _NPAD + s * _SL, _SL)])


@functools.partial(
    pl.kernel,
    out_type=jax.ShapeDtypeStruct((2 * _NPAD,), jnp.float32),
    mesh=_mesh,
    scratch_types=[
        pltpu.VMEM((2, 2, _ROWS, _RB), jnp.int32),
        pltpu.VMEM((_ROWS, _RB), jnp.float32),
        pltpu.VMEM((_SL,), jnp.float32),
        pltpu.VMEM_SHARED((_NPAD,), jnp.float32),
        pltpu.VMEM_SHARED((_NPAD,), jnp.float32),
        pltpu.SemaphoreType.DMA,
        pltpu.SemaphoreType.DMA,
        pltpu.SemaphoreType.DMA,
        pltpu.SemaphoreType.DMA,
    ],
)
def _agg_pass(edges, u, out, ebuf, vals, zbuf, utab, acc,
              esem0, esem1, gsem, ssem):
    c = lax.axis_index("c")
    s = lax.axis_index("s")
    w = s * 2 + c
    esems = (esem0, esem1)

    def _fill_zero(i, carry):
        zbuf[pl.ds(i * 16, 16)] = jnp.zeros((16,), jnp.float32)
        return carry

    lax.fori_loop(0, _SL // 16, _fill_zero, 0)
    pltpu.sync_copy(zbuf, acc.at[pl.ds(s * _SL, _SL)])
    pltpu.sync_copy(u.at[pl.ds(s * _SL, _SL)], zbuf)
    pltpu.sync_copy(zbuf, utab.at[pl.ds(s * _SL, _SL)])
    plsc.subcore_barrier()

    def _edges_dma(cid, b):
        return pltpu.make_async_copy(
            edges.at[:, pl.ds(cid * _ROWS, _ROWS), :], ebuf.at[b], esems[b])

    for b in range(2):
        cid = w + b * _NW

        @pl.when(cid < _NCH)
        def _():
            _edges_dma(cid, b).start()

    def _chunk(j, carry):
        for b in range(2):
            cid = w + (2 * j + b) * _NW

            @pl.when(cid < _NCH)
            def _():
                _edges_dma(cid, b).wait()
                gd = [
                    pltpu.async_copy(utab.at[ebuf.at[b, 0, r]], vals.at[r],
                                     gsem)
                    for r in range(_ROWS)
                ]
                for d in gd:
                    d.wait()
                sd = [
                    pltpu.async_copy(vals.at[r], acc.at[ebuf.at[b, 1, r]],
                                     ssem, add=True)
                    for r in range(_ROWS)
                ]
                for d in sd:
                    d.wait()
                nid = cid + 2 * _NW

                @pl.when(nid < _NCH)
                def _():
                    _edges_dma(nid, b).start()

        return carry

    lax.fori_loop(0, (_NITER + 1) // 2, _chunk, 0)
    plsc.subcore_barrier()
    pltpu.sync_copy(acc.at[pl.ds(s * _SL, _SL)], zbuf)
    pltpu.sync_copy(zbuf, out.at[pl.ds(c * _NPAD + s * _SL, _SL)])


def _e1_body(parts, xp, u1, dinv, xod):
    deg = parts[0] + parts[1] + 1.0
    di = lax.rsqrt(deg)
    dinv[...] = di
    u1[...] = di * xp[...]
    xod[...] = xp[...] * di * di


_e1 = pl.pallas_call(
    _e1_body,
    out_shape=(
        jax.ShapeDtypeStruct((_RPT, 128), jnp.float32),
        jax.ShapeDtypeStruct((_RPT, 128), jnp.float32),
        jax.ShapeDtypeStruct((_RPT, 128), jnp.float32),
    ),
)


def _e2_body(coef, parts, dinv, xod, u2, selfout):
    di = dinv[...]
    agg1 = di * (parts[0] + parts[1]) + xod[...]
    h1a = jnp.maximum(agg1 * coef[0] + coef[2], 0.0)
    h1b = jnp.maximum(agg1 * coef[1] + coef[3], 0.0)
    h2 = h1a * coef[4] + h1b * coef[5]
    u2[...] = di * h2
    selfout[...] = h2 * di * di + coef[6]


_e2 = pl.pallas_call(
    _e2_body,
    in_specs=[
        pl.BlockSpec(memory_space=pltpu.SMEM),
        pl.BlockSpec(memory_space=pltpu.VMEM),
        pl.BlockSpec(memory_space=pltpu.VMEM),
        pl.BlockSpec(memory_space=pltpu.VMEM),
    ],
    out_shape=(
        jax.ShapeDtypeStruct((_RPT, 128), jnp.float32),
        jax.ShapeDtypeStruct((_RPT, 128), jnp.float32),
    ),
)


def _e3_body(parts, dinv, selfout, out):
    out[...] = dinv[...] * (parts[0] + parts[1]) + selfout[...]


_e3 = pl.pallas_call(
    _e3_body,
    out_shape=jax.ShapeDtypeStruct((_RPT, 128), jnp.float32),
)


def kernel(x, edge_index, W1, b1, W2, b2):
    xf = x.reshape(_N).astype(jnp.float32)
    xp = jnp.pad(xf, (0, _NPAD - _N)).reshape(_RPT, 128)
    ei = edge_index.astype(jnp.int32).reshape(2, _EROWS, _RB)
    coef = jnp.stack(
        [W1[0, 0], W1[0, 1], b1[0], b1[1], W2[0, 0], W2[1, 0], b2[0],
         jnp.float32(0.0)]
    )
    parts0 = _deg_pass(ei)
    u1, dinv, xod = _e1(parts0.reshape(2, _RPT, 128), xp)
    parts1 = _agg_pass(ei, u1.reshape(_NPAD))
    u2, selfout = _e2(coef, parts1.reshape(2, _RPT, 128), dinv, xod)
    parts2 = _agg_pass(ei, u2.reshape(_NPAD))
    outp = _e3(parts2.reshape(2, _RPT, 128), dinv, selfout)
    return outp.reshape(_NPAD)[:_N].reshape(_N, 1)


# speedup vs baseline: 596.9112x; 1.9579x over previous
import functools

import jax
import jax.numpy as jnp
from jax import lax
from jax.experimental import pallas as pl
from jax.experimental.pallas import tpu as pltpu
from jax.experimental.pallas import tpu_sc as plsc

_N = 100000
_E = 6400000
_RB = 128
_ROWS = 16
_CH = _RB * _ROWS
_NCH = _E // _CH
_EROWS = _E // _RB
_NW = 32
_NITER = -(-_NCH // _NW)
_NPAD = 100096
_RPT = _NPAD // 128
_SL = _NPAD // 16

_mesh = plsc.VectorSubcoreMesh(core_axis_name="c", subcore_axis_name="s")


@functools.partial(
    pl.kernel,
    out_type=jax.ShapeDtypeStruct((2 * _NPAD,), jnp.float32),
    mesh=_mesh,
    scratch_types=[
        pltpu.VMEM((2, _ROWS, _RB), jnp.int32),
        pltpu.VMEM((_RB,), jnp.float32),
        pltpu.VMEM((_SL,), jnp.float32),
        pltpu.VMEM_SHARED((_NPAD,), jnp.float32),
        pltpu.SemaphoreType.DMA,
        pltpu.SemaphoreType.DMA,
        pltpu.SemaphoreType.DMA,
    ],
)
def _deg_pass(edges, out, dstbuf, ones, zbuf, acc, esem0, esem1, ssem):
    c = lax.axis_index("c")
    s = lax.axis_index("s")
    w = s * 2 + c
    esems = (esem0, esem1)

    def _fill_zero(i, carry):
        zbuf[pl.ds(i * 16, 16)] = jnp.zeros((16,), jnp.float32)
        return carry

    lax.fori_loop(0, _SL // 16, _fill_zero, 0)

    def _fill_one(i, carry):
        ones[pl.ds(i * 16, 16)] = jnp.full((16,), 1.0, jnp.float32)
        return carry

    lax.fori_loop(0, _RB // 16, _fill_one, 0)
    pltpu.sync_copy(zbuf, acc.at[pl.ds(s * _SL, _SL)])
    plsc.subcore_barrier()

    def _edges_dma(cid, b):
        return pltpu.make_async_copy(
            edges.at[1, pl.ds(cid * _ROWS, _ROWS), :], dstbuf.at[b], esems[b])

    for b in range(2):
        cid = w + b * _NW

        @pl.when(cid < _NCH)
        def _():
            _edges_dma(cid, b).start()

    def _chunk(j, carry):
        for b in range(2):
            cid = w + (2 * j + b) * _NW

            @pl.when(cid < _NCH)
            def _():
                _edges_dma(cid, b).wait()
                descs = [
                    pltpu.async_copy(ones, acc.at[dstbuf.at[b, r]], ssem,
                                     add=True)
                    for r in range(_ROWS)
                ]
                for d in descs:
                    d.wait()
                nid = cid + 2 * _NW

                @pl.when(nid < _NCH)
                def _():
                    _edges_dma(nid, b).start()

        return carry

    lax.fori_loop(0, (_NITER + 1) // 2, _chunk, 0)
    plsc.subcore_barrier()
    pltpu.sync_copy(acc.at[pl.ds(s * _SL, _SL)], zbuf)
    pltpu.sync_copy(zbuf, out.at[pl.ds(c * _NPAD + s * _SL, _SL)])


@functools.partial(
    pl.kernel,
    out_type=jax.ShapeDtypeStruct((2 * _NPAD,), jnp.float32),
    mesh=_mesh,
    scratch_types=[
        pltpu.VMEM((3, 2, _ROWS, _RB), jnp.int32),
        pltpu.VMEM((3, _ROWS, _RB), jnp.float32),
        pltpu.VMEM((_SL,), jnp.float32),
        pltpu.VMEM((_NPAD,), jnp.float32),
        pltpu.VMEM_SHARED((_NPAD,), jnp.float32),
        pltpu.SemaphoreType.DMA,
        pltpu.SemaphoreType.DMA,
        pltpu.SemaphoreType.DMA,
        pltpu.SemaphoreType.DMA,
    ],
    compiler_params=pltpu.CompilerParams(needs_layout_passes=False),
)
def _agg_pass(edges, u, out, ebuf, vals, zbuf, utab, acc,
              esem0, esem1, esem2, ssem):
    c = lax.axis_index("c")
    s = lax.axis_index("s")
    w = s * 2 + c
    esems = (esem0, esem1, esem2)

    def _fill_zero(i, carry):
        zbuf[pl.ds(i * 16, 16)] = jnp.zeros((16,), jnp.float32)
        return carry

    lax.fori_loop(0, _SL // 16, _fill_zero, 0)
    pltpu.sync_copy(zbuf, acc.at[pl.ds(s * _SL, _SL)])
    pltpu.sync_copy(u, utab)
    plsc.subcore_barrier()

    def _edges_dma(cid, m):
        return pltpu.make_async_copy(
            edges.at[:, pl.ds(cid * _ROWS, _ROWS), :], ebuf.at[m], esems[m])

    def _scat(m, r):
        return pltpu.make_async_copy(
            vals.at[m, r], acc.at[ebuf.at[m, 1, r]], ssem)

    @pl.when(w < _NCH)
    def _():
        _edges_dma(w, 0).start()

    def _section(j, carry):
        for b in range(3):
            k = 3 * j + b
            cid = w + k * _NW
            m = b
            m2 = (b - 2) % 3
            m1 = (b + 1) % 3
            cid2 = cid - 2 * _NW
            cid1 = cid + _NW

            @pl.when((k >= 2) & (cid2 < _NCH))
            def _():
                for r in range(_ROWS):
                    _scat(m2, r).wait()

            @pl.when(cid1 < _NCH)
            def _():
                _edges_dma(cid1, m1).start()

            @pl.when(cid < _NCH)
            def _():
                _edges_dma(cid, m).wait()
                for rr in range(_ROWS):
                    for l in range(_RB // 16):
                        sidx = ebuf[m, 0, rr, pl.ds(l * 16, 16)]
                        vals[m, rr, pl.ds(l * 16, 16)] = plsc.load_gather(
                            utab, [sidx])
                for r in range(_ROWS):
                    pltpu.async_copy(vals.at[m, r], acc.at[ebuf.at[m, 1, r]],
                                     ssem, add=True)

        return carry

    lax.fori_loop(0, (_NITER + 2 + 2) // 3, _section, 0)
    plsc.subcore_barrier()
    pltpu.sync_copy(acc.at[pl.ds(s * _SL, _SL)], zbuf)
    pltpu.sync_copy(zbuf, out.at[pl.ds(c * _NPAD + s * _SL, _SL)])


def _e1_body(parts, xp, u1, dinv, xod):
    deg = parts[0] + parts[1] + 1.0
    di = lax.rsqrt(deg)
    dinv[...] = di
    u1[...] = di * xp[...]
    xod[...] = xp[...] * di * di


_e1 = pl.pallas_call(
    _e1_body,
    out_shape=(
        jax.ShapeDtypeStruct((_RPT, 128), jnp.float32),
        jax.ShapeDtypeStruct((_RPT, 128), jnp.float32),
        jax.ShapeDtypeStruct((_RPT, 128), jnp.float32),
    ),
)


def _e2_body(coef, parts, dinv, xod, u2, selfout):
    di = dinv[...]
    agg1 = di * (parts[0] + parts[1]) + xod[...]
    h1a = jnp.maximum(agg1 * coef[0] + coef[2], 0.0)
    h1b = jnp.maximum(agg1 * coef[1] + coef[3], 0.0)
    h2 = h1a * coef[4] + h1b * coef[5]
    u2[...] = di * h2
    selfout[...] = h2 * di * di + coef[6]


_e2 = pl.pallas_call(
    _e2_body,
    in_specs=[
        pl.BlockSpec(memory_space=pltpu.SMEM),
        pl.BlockSpec(memory_space=pltpu.VMEM),
        pl.BlockSpec(memory_space=pltpu.VMEM),
        pl.BlockSpec(memory_space=pltpu.VMEM),
    ],
    out_shape=(
        jax.ShapeDtypeStruct((_RPT, 128), jnp.float32),
        jax.ShapeDtypeStruct((_RPT, 128), jnp.float32),
    ),
)


def _e3_body(parts, dinv, selfout, out):
    out[...] = dinv[...] * (parts[0] + parts[1]) + selfout[...]


_e3 = pl.pallas_call(
    _e3_body,
    out_shape=jax.ShapeDtypeStruct((_RPT, 128), jnp.float32),
)


def kernel(x, edge_index, W1, b1, W2, b2):
    xf = x.reshape(_N).astype(jnp.float32)
    xp = jnp.pad(xf, (0, _NPAD - _N)).reshape(_RPT, 128)
    ei = edge_index.astype(jnp.int32).reshape(2, _EROWS, _RB)
    coef = jnp.stack(
        [W1[0, 0], W1[0, 1], b1[0], b1[1], W2[0, 0], W2[1, 0], b2[0],
         jnp.float32(0.0)]
    )
    parts0 = _deg_pass(ei)
    u1, dinv, xod = _e1(parts0.reshape(2, _RPT, 128), xp)
    parts1 = _agg_pass(ei, u1.reshape(_NPAD))
    u2, selfout = _e2(coef, parts1.reshape(2, _RPT, 128), dinv, xod)
    parts2 = _agg_pass(ei, u2.reshape(_NPAD))
    outp = _e3(parts2.reshape(2, _RPT, 128), dinv, selfout)
    return outp.reshape(_NPAD)[:_N].reshape(_N, 1)


# speedup vs baseline: 611.3721x; 1.0242x over previous
import functools

import jax
import jax.numpy as jnp
from jax import lax
from jax.experimental import pallas as pl
from jax.experimental.pallas import tpu as pltpu
from jax.experimental.pallas import tpu_sc as plsc

_N = 100000
_E = 6400000
_RB = 128
_ROWS = 16
_CH = _RB * _ROWS
_NCH = _E // _CH
_EROWS = _E // _RB
_NW = 32
_NITER = -(-_NCH // _NW)
_NPAD = 100096
_RPT = _NPAD // 128
_SL = _NPAD // 16

_mesh = plsc.VectorSubcoreMesh(core_axis_name="c", subcore_axis_name="s")


@functools.partial(
    pl.kernel,
    out_type=jax.ShapeDtypeStruct((2 * _NPAD,), jnp.float32),
    mesh=_mesh,
    scratch_types=[
        pltpu.VMEM((3, _ROWS, _RB), jnp.int32),
        pltpu.VMEM((_RB,), jnp.float32),
        pltpu.VMEM((_SL,), jnp.float32),
        pltpu.VMEM_SHARED((_NPAD,), jnp.float32),
        pltpu.SemaphoreType.DMA,
        pltpu.SemaphoreType.DMA,
        pltpu.SemaphoreType.DMA,
        pltpu.SemaphoreType.DMA,
    ],
    compiler_params=pltpu.CompilerParams(use_tc_tiling_on_sc=True),
)
def _deg_pass(edges, out, dstbuf, ones, zbuf, acc, esem0, esem1, esem2, ssem):
    c = lax.axis_index("c")
    s = lax.axis_index("s")
    w = s * 2 + c
    esems = (esem0, esem1, esem2)

    def _fill_zero(i, carry):
        zbuf[pl.ds(i * 16, 16)] = jnp.zeros((16,), jnp.float32)
        return carry

    lax.fori_loop(0, _SL // 16, _fill_zero, 0)

    def _fill_one(i, carry):
        ones[pl.ds(i * 16, 16)] = jnp.full((16,), 1.0, jnp.float32)
        return carry

    lax.fori_loop(0, _RB // 16, _fill_one, 0)
    pltpu.sync_copy(zbuf, acc.at[pl.ds(s * _SL, _SL)])
    plsc.subcore_barrier()

    def _edges_dma(cid, m):
        return pltpu.make_async_copy(
            edges.at[1, pl.ds(cid * _ROWS, _ROWS), :], dstbuf.at[m], esems[m])

    @pl.when(w < _NCH)
    def _():
        _edges_dma(w, 0).start()

    def _section(j, carry):
        for b in range(3):
            k = 3 * j + b
            cid = w + k * _NW
            m2 = (b - 2) % 3
            m1 = (b + 1) % 3
            cid2 = cid - 2 * _NW
            cid1 = cid + _NW

            @pl.when((k >= 2) & (cid2 < _NCH))
            def _():
                for r in range(_ROWS):
                    pltpu.make_async_copy(
                        ones, acc.at[dstbuf.at[m2, r]], ssem).wait()

            @pl.when(cid1 < _NCH)
            def _():
                _edges_dma(cid1, m1).start()

            @pl.when(cid < _NCH)
            def _():
                _edges_dma(cid, b).wait()
                for r in range(_ROWS):
                    pltpu.async_copy(ones, acc.at[dstbuf.at[b, r]], ssem,
                                     add=True)

        return carry

    lax.fori_loop(0, (_NITER + 2 + 2) // 3, _section, 0)
    plsc.subcore_barrier()
    pltpu.sync_copy(acc.at[pl.ds(s * _SL, _SL)], zbuf)
    pltpu.sync_copy(zbuf, out.at[pl.ds(c * _NPAD + s * _SL, _SL)])


@functools.partial(
    pl.kernel,
    out_type=jax.ShapeDtypeStruct((2 * _NPAD,), jnp.float32),
    mesh=_mesh,
    scratch_types=[
        pltpu.VMEM((3, 2, _ROWS, _RB), jnp.int32),
        pltpu.VMEM((3, _ROWS, _RB), jnp.float32),
        pltpu.VMEM((_SL,), jnp.float32),
        pltpu.VMEM((_NPAD,), jnp.float32),
        pltpu.VMEM_SHARED((_NPAD,), jnp.float32),
        pltpu.SemaphoreType.DMA,
        pltpu.SemaphoreType.DMA,
        pltpu.SemaphoreType.DMA,
        pltpu.SemaphoreType.DMA,
    ],
    compiler_params=pltpu.CompilerParams(needs_layout_passes=False,
                                         use_tc_tiling_on_sc=True),
)
def _agg_pass(edges, u, out, ebuf, vals, zbuf, utab, acc,
              esem0, esem1, esem2, ssem):
    c = lax.axis_index("c")
    s = lax.axis_index("s")
    w = s * 2 + c
    esems = (esem0, esem1, esem2)

    def _fill_zero(i, carry):
        zbuf[pl.ds(i * 16, 16)] = jnp.zeros((16,), jnp.float32)
        return carry

    lax.fori_loop(0, _SL // 16, _fill_zero, 0)
    pltpu.sync_copy(zbuf, acc.at[pl.ds(s * _SL, _SL)])
    pltpu.sync_copy(u, utab)
    plsc.subcore_barrier()

    def _edges_dma(cid, m):
        return pltpu.make_async_copy(
            edges.at[:, pl.ds(cid * _ROWS, _ROWS), :], ebuf.at[m], esems[m])

    def _scat(m, r):
        return pltpu.make_async_copy(
            vals.at[m, r], acc.at[ebuf.at[m, 1, r]], ssem)

    @pl.when(w < _NCH)
    def _():
        _edges_dma(w, 0).start()

    def _section(j, carry):
        for b in range(3):
            k = 3 * j + b
            cid = w + k * _NW
            m = b
            m2 = (b - 2) % 3
            m1 = (b + 1) % 3
            cid2 = cid - 2 * _NW
            cid1 = cid + _NW

            @pl.when((k >= 2) & (cid2 < _NCH))
            def _():
                for r in range(_ROWS):
                    _scat(m2, r).wait()

            @pl.when(cid1 < _NCH)
            def _():
                _edges_dma(cid1, m1).start()

            @pl.when(cid < _NCH)
            def _():
                _edges_dma(cid, m).wait()
                for rr in range(_ROWS):
                    for l in range(_RB // 16):
                        sidx = ebuf[m, 0, rr, pl.ds(l * 16, 16)]
                        vals[m, rr, pl.ds(l * 16, 16)] = plsc.load_gather(
                            utab, [sidx])
                for r in range(_ROWS):
                    pltpu.async_copy(vals.at[m, r], acc.at[ebuf.at[m, 1, r]],
                                     ssem, add=True)

        return carry

    lax.fori_loop(0, (_NITER + 2 + 2) // 3, _section, 0)
    plsc.subcore_barrier()
    pltpu.sync_copy(acc.at[pl.ds(s * _SL, _SL)], zbuf)
    pltpu.sync_copy(zbuf, out.at[pl.ds(c * _NPAD + s * _SL, _SL)])


def _e1_body(parts, xp, u1, dinv, xod):
    deg = parts[0] + parts[1] + 1.0
    di = lax.rsqrt(deg)
    dinv[...] = di
    u1[...] = di * xp[...]
    xod[...] = xp[...] * di * di


_e1 = pl.pallas_call(
    _e1_body,
    out_shape=(
        jax.ShapeDtypeStruct((_RPT, 128), jnp.float32),
        jax.ShapeDtypeStruct((_RPT, 128), jnp.float32),
        jax.ShapeDtypeStruct((_RPT, 128), jnp.float32),
    ),
)


def _e2_body(coef, parts, dinv, xod, u2, selfout):
    di = dinv[...]
    agg1 = di * (parts[0] + parts[1]) + xod[...]
    h1a = jnp.maximum(agg1 * coef[0] + coef[2], 0.0)
    h1b = jnp.maximum(agg1 * coef[1] + coef[3], 0.0)
    h2 = h1a * coef[4] + h1b * coef[5]
    u2[...] = di * h2
    selfout[...] = h2 * di * di + coef[6]


_e2 = pl.pallas_call(
    _e2_body,
    in_specs=[
        pl.BlockSpec(memory_space=pltpu.SMEM),
        pl.BlockSpec(memory_space=pltpu.VMEM),
        pl.BlockSpec(memory_space=pltpu.VMEM),
        pl.BlockSpec(memory_space=pltpu.VMEM),
    ],
    out_shape=(
        jax.ShapeDtypeStruct((_RPT, 128), jnp.float32),
        jax.ShapeDtypeStruct((_RPT, 128), jnp.float32),
    ),
)


def _e3_body(parts, dinv, selfout, out):
    out[...] = dinv[...] * (parts[0] + parts[1]) + selfout[...]


_e3 = pl.pallas_call(
    _e3_body,
    out_shape=jax.ShapeDtypeStruct((_RPT, 128), jnp.float32),
)


def kernel(x, edge_index, W1, b1, W2, b2):
    xf = x.reshape(_N).astype(jnp.float32)
    xp = jnp.pad(xf, (0, _NPAD - _N)).reshape(_RPT, 128)
    ei = edge_index.astype(jnp.int32).reshape(2, _EROWS, _RB)
    coef = jnp.stack(
        [W1[0, 0], W1[0, 1], b1[0], b1[1], W2[0, 0], W2[1, 0], b2[0],
         jnp.float32(0.0)]
    )
    parts0 = _deg_pass(ei)
    u1, dinv, xod = _e1(parts0.reshape(2, _RPT, 128), xp)
    parts1 = _agg_pass(ei, u1.reshape(_NPAD))
    u2, selfout = _e2(coef, parts1.reshape(2, _RPT, 128), dinv, xod)
    parts2 = _agg_pass(ei, u2.reshape(_NPAD))
    outp = _e3(parts2.reshape(2, _RPT, 128), dinv, selfout)
    return outp.reshape(_NPAD)[:_N].reshape(_N, 1)
